# baseline (speedup 1.0000x reference)
import functools

import jax
import jax.numpy as jnp
from jax import lax
from jax.experimental import pallas as pl
from jax.experimental.pallas import tpu as pltpu
from jax.experimental.pallas import tpu_sc as plsc

B, S, D = 1, 2048, 1024
E, K, DFF = 8, 2, 1024

TM = 256
NP = S * K + E * TM
NTILES = NP // TM
CHUNK = 256
NW = 32
TOK_W = S // NW
TOK_C = 32
D2 = D // 2


_HI_MASK = -65536


def _bf16_bits(v):
    return lax.shift_right_logical(
        lax.bitcast_convert_type(v, jnp.int32) + jnp.int32(0x8000), 16)


def _routing_kernel(scores_ref, x_ref, pos0_ref, pos1_ref, pr0_ref, pr1_ref,
                    emap_ref, xpk_ref):
    xv = x_ref[...]
    lo = _bf16_bits(xv[:, :D2])
    hi = _bf16_bits(xv[:, D2:])
    xpk_ref[...] = jnp.bitwise_or(lo, lax.shift_left(hi, 16))
    scores = scores_ref[...]
    iota_e = lax.broadcasted_iota(jnp.int32, (S, E), 1)
    m1 = jnp.max(scores, axis=1, keepdims=True)
    i1 = jnp.min(jnp.where(scores == m1, iota_e, E), axis=1, keepdims=True)
    masked = jnp.where(iota_e == i1, -jnp.inf, scores)
    m2 = jnp.max(masked, axis=1, keepdims=True)
    i2 = jnp.min(jnp.where(masked == m2, iota_e, E), axis=1, keepdims=True)
    e2 = jnp.exp(m2 - m1)
    denom = 1.0 + e2
    pr0_ref[...] = 1.0 / denom
    pr1_ref[...] = e2 / denom

    one1 = (iota_e == i1).astype(jnp.float32)
    one2 = (iota_e == i2).astype(jnp.float32)
    osum = one1 + one2

    r = lax.broadcasted_iota(jnp.int32, (CHUNK, CHUNK), 0)
    c = lax.broadcasted_iota(jnp.int32, (CHUNK, CHUNK), 1)
    tril = (r > c).astype(jnp.float32)

    run = jnp.zeros((1, E), jnp.float32)
    rank1_chunks, rank2_chunks = [], []
    for ci in range(S // CHUNK):
        blk = osum[ci * CHUNK:(ci + 1) * CHUNK]
        cum = lax.dot_general(tril, blk, (((1,), (0,)), ((), ())),
                              preferred_element_type=jnp.float32) + run
        o1c = one1[ci * CHUNK:(ci + 1) * CHUNK]
        o2c = one2[ci * CHUNK:(ci + 1) * CHUNK]
        rank1_chunks.append(jnp.sum(cum * o1c, axis=1, keepdims=True))
        rank2_chunks.append(jnp.sum(cum * o2c, axis=1, keepdims=True))
        run = run + jnp.sum(blk, axis=0, keepdims=True)
    rank1 = jnp.concatenate(rank1_chunks, axis=0)
    rank2 = jnp.concatenate(rank2_chunks, axis=0)

    counts = run
    asz = jnp.floor((counts + (TM - 1)) * (1.0 / TM)) * TM
    ru = lax.broadcasted_iota(jnp.int32, (E, E), 0)
    cu = lax.broadcasted_iota(jnp.int32, (E, E), 1)
    upper = (ru < cu).astype(jnp.float32)
    starts = lax.dot_general(asz, upper, (((1,), (0,)), ((), ())),
                             preferred_element_type=jnp.float32)

    pos0 = jnp.sum(starts * one1, axis=1, keepdims=True) + rank1
    pos1 = jnp.sum(starts * one2, axis=1, keepdims=True) + rank2
    pos0_ref[...] = pos0.astype(jnp.int32)
    pos1_ref[...] = pos1.astype(jnp.int32)

    t_start = lax.broadcasted_iota(jnp.int32, (1, NTILES), 1).astype(
        jnp.float32) * TM
    lane8 = lax.broadcasted_iota(jnp.int32, (1, E), 1)
    acc = jnp.zeros((1, NTILES), jnp.float32)
    for e in range(E):
        s_e = jnp.sum(jnp.where(lane8 == e, starts, 0.0))
        acc = acc + (t_start >= s_e).astype(jnp.float32)
    emap_ref[...] = acc.astype(jnp.int32) - 1


def _dispatch_body(x_hbm, pos0_hbm, pos1_hbm, xs_hbm, rows_v, idx0_v, idx1_v,
                   sem0, sem1):
    wid = lax.axis_index("s") * 2 + lax.axis_index("c")
    base = wid * TOK_W
    pltpu.sync_copy(x_hbm.at[pl.ds(base, TOK_W)], rows_v)
    pltpu.sync_copy(pos0_hbm.at[pl.ds(base, TOK_W)], idx0_v)
    pltpu.sync_copy(pos1_hbm.at[pl.ds(base, TOK_W)], idx1_v)
    c0 = pltpu.async_copy(rows_v, xs_hbm.at[idx0_v], sem0)
    c1 = pltpu.async_copy(rows_v, xs_hbm.at[idx1_v], sem1)
    c0.wait()
    c1.wait()


def _ffn_kernel(emap_ref, xs_ref, w1_ref, w2_ref, w3_ref, out_ref,
                w1b_ref, w2b_ref, w3b_ref):
    i = pl.program_id(0)
    changed = jnp.logical_or(
        i == 0, emap_ref[i] != emap_ref[jnp.maximum(i - 1, 0)])

    @pl.when(changed)
    def _():
        w1b_ref[...] = w1_ref[0].astype(jnp.bfloat16)
        w2b_ref[...] = w2_ref[0].astype(jnp.bfloat16)
        w3b_ref[...] = w3_ref[0].astype(jnp.bfloat16)

    wv = xs_ref[...]
    xlo = lax.bitcast_convert_type(lax.shift_left(wv, 16), jnp.float32)
    xhi = lax.bitcast_convert_type(
        jnp.bitwise_and(wv, jnp.int32(_HI_MASK)), jnp.float32)
    xb = jnp.concatenate([xlo, xhi], axis=1).astype(jnp.bfloat16)
    a = lax.dot_general(xb, w1b_ref[...], (((1,), (1,)), ((), ())),
                        preferred_element_type=jnp.float32)
    b = lax.dot_general(xb, w2b_ref[...], (((1,), (1,)), ((), ())),
                        preferred_element_type=jnp.float32)
    h = (jax.nn.silu(a) * b).astype(jnp.bfloat16)
    o = lax.dot_general(h, w3b_ref[...], (((1,), (1,)), ((), ())),
                        preferred_element_type=jnp.float32)
    out_ref[...] = jnp.bitwise_or(
        _bf16_bits(o[:, :D2]), lax.shift_left(_bf16_bits(o[:, D2:]), 16))


_ffn_grid_spec = pltpu.PrefetchScalarGridSpec(
    num_scalar_prefetch=1,
    grid=(NTILES,),
    in_specs=[
        pl.BlockSpec((TM, D2), lambda i, em: (i, 0)),
        pl.BlockSpec((1, DFF, D), lambda i, em: (em[i], 0, 0)),
        pl.BlockSpec((1, DFF, D), lambda i, em: (em[i], 0, 0)),
        pl.BlockSpec((1, D, DFF), lambda i, em: (em[i], 0, 0)),
    ],
    out_specs=pl.BlockSpec((TM, D2), lambda i, em: (i, 0)),
    scratch_shapes=[
        pltpu.VMEM((DFF, D), jnp.bfloat16),
        pltpu.VMEM((DFF, D), jnp.bfloat16),
        pltpu.VMEM((D, DFF), jnp.bfloat16),
    ],
)


def _bf16_pair_fma(r0, r1, p0, p1):
    hi_mask = jnp.int32(_HI_MASK)
    _f = lambda v: lax.bitcast_convert_type(v, jnp.float32)
    lo0 = _f(lax.shift_left(r0, 16))
    lo1 = _f(lax.shift_left(r1, 16))
    hi0 = _f(jnp.bitwise_and(r0, hi_mask))
    hi1 = _f(jnp.bitwise_and(r1, hi_mask))
    return lo0 * p0 + lo1 * p1, hi0 * p0 + hi1 * p1


def _combine_body(outp_hbm, pos0_hbm, pos1_hbm, pr0_hbm, pr1_hbm, y_hbm,
                  rows0_a, rows1_a, rows0_b, rows1_b, y_v,
                  idx0_v, idx1_v, pr0_v, pr1_v, sem0, sem1, sem_y):
    wid = lax.axis_index("s") * 2 + lax.axis_index("c")
    base = wid * TOK_W
    pltpu.sync_copy(pos0_hbm.at[pl.ds(base, TOK_W)], idx0_v)
    pltpu.sync_copy(pos1_hbm.at[pl.ds(base, TOK_W)], idx1_v)
    pltpu.sync_copy(pr0_hbm.at[pl.ds(base, TOK_W)], pr0_v)
    pltpu.sync_copy(pr1_hbm.at[pl.ds(base, TOK_W)], pr1_v)

    rows0 = [rows0_a, rows0_b]
    rows1 = [rows1_a, rows1_b]

    def issue(c):
        i0 = idx0_v.at[pl.ds(c * TOK_C, TOK_C)]
        i1 = idx1_v.at[pl.ds(c * TOK_C, TOK_C)]
        h0 = pltpu.async_copy(outp_hbm.at[i0], rows0[c], sem0)
        h1 = pltpu.async_copy(outp_hbm.at[i1], rows1[c], sem1)
        return h0, h1

    nch = TOK_W // TOK_C
    handles = {0: issue(0)}
    ywrite = None
    for c in range(nch):
        handles[c][0].wait()
        handles[c][1].wait()
        if c + 1 < nch:
            handles[c + 1] = issue(c + 1)
        if ywrite is not None:
            ywrite.wait()
        for g in range(TOK_C // 16):
            prv0 = pr0_v[pl.ds(c * TOK_C + g * 16, 16)]
            prv1 = pr1_v[pl.ds(c * TOK_C + g * 16, 16)]
            for tl in range(16):
                t = g * 16 + tl
                lane = jnp.full((16,), tl, jnp.int32)
                p0 = prv0[lane]
                p1 = prv1[lane]

                def body(ci, carry, t=t, p0=p0, p1=p1, c=c):
                    sl = pl.ds(ci * 16, 16)
                    ylo, yhi = _bf16_pair_fma(
                        rows0[c][t, sl], rows1[c][t, sl], p0, p1)
                    y_v[t, pl.ds(ci * 16, 16)] = ylo
                    y_v[t, pl.ds(D2 + ci * 16, 16)] = yhi
                    return carry

                lax.fori_loop(0, D2 // 16, body, 0, unroll=4)
        ywrite = pltpu.async_copy(
            y_v, y_hbm.at[pl.ds(base + c * TOK_C, TOK_C)], sem_y)
    ywrite.wait()


@functools.cache
def _sc_kernels():
    mesh = plsc.VectorSubcoreMesh(core_axis_name="c", subcore_axis_name="s")
    dispatch = pl.kernel(
        _dispatch_body,
        out_type=jax.ShapeDtypeStruct((NP, D2), jnp.int32),
        mesh=mesh,
        scratch_types=[
            pltpu.VMEM((TOK_W, D2), jnp.int32),
            pltpu.VMEM((TOK_W,), jnp.int32),
            pltpu.VMEM((TOK_W,), jnp.int32),
            pltpu.SemaphoreType.DMA,
            pltpu.SemaphoreType.DMA,
        ],
    )
    combine = pl.kernel(
        _combine_body,
        out_type=jax.ShapeDtypeStruct((S, D), jnp.float32),
        mesh=mesh,
        scratch_types=[
            pltpu.VMEM((TOK_C, D2), jnp.int32),
            pltpu.VMEM((TOK_C, D2), jnp.int32),
            pltpu.VMEM((TOK_C, D2), jnp.int32),
            pltpu.VMEM((TOK_C, D2), jnp.int32),
            pltpu.VMEM((TOK_C, D), jnp.float32),
            pltpu.VMEM((TOK_W,), jnp.int32),
            pltpu.VMEM((TOK_W,), jnp.int32),
            pltpu.VMEM((TOK_W,), jnp.float32),
            pltpu.VMEM((TOK_W,), jnp.float32),
            pltpu.SemaphoreType.DMA,
            pltpu.SemaphoreType.DMA,
            pltpu.SemaphoreType.DMA,
        ],
    )
    return dispatch, combine


@jax.jit
def kernel(x, Wg, W1, W2, W3):
    x2d = x.reshape(S, D)
    scores = jnp.einsum('bsd,ed->bse', x, Wg).reshape(S, E)

    pos0, pos1, pr0, pr1, emap, xpk = pl.pallas_call(
        _routing_kernel,
        out_shape=[
            jax.ShapeDtypeStruct((S, 1), jnp.int32),
            jax.ShapeDtypeStruct((S, 1), jnp.int32),
            jax.ShapeDtypeStruct((S, 1), jnp.float32),
            jax.ShapeDtypeStruct((S, 1), jnp.float32),
            jax.ShapeDtypeStruct((1, NTILES), jnp.int32),
            jax.ShapeDtypeStruct((S, D2), jnp.int32),
        ],
    )(scores, x2d)
    pos0f = pos0.reshape(S)
    pos1f = pos1.reshape(S)

    dispatch, combine = _sc_kernels()
    xs_pk = dispatch(xpk, pos0f, pos1f)
    outp_pk = pl.pallas_call(
        _ffn_kernel,
        grid_spec=_ffn_grid_spec,
        out_shape=jax.ShapeDtypeStruct((NP, D2), jnp.int32),
    )(emap.reshape(NTILES), xs_pk, W1, W2, W3)
    y = combine(outp_pk, pos0f, pos1f, pr0.reshape(S), pr1.reshape(S))
    return y.reshape(B, S, D)


# speedup vs baseline: 1.0836x; 1.0836x over previous
import functools

import jax
import jax.numpy as jnp
from jax import lax
from jax.experimental import pallas as pl
from jax.experimental.pallas import tpu as pltpu
from jax.experimental.pallas import tpu_sc as plsc

B, S, D = 1, 2048, 1024
E, K, DFF = 8, 2, 1024

TM = 256
NP = S * K + E * TM
NTILES = NP // TM
CHUNK = 256
NW = 32
TOK_W = S // NW
TOK_C = 32
D2 = D // 2


_HI_MASK = -65536


def _bf16_bits(v):
    return lax.shift_right_logical(
        lax.bitcast_convert_type(v, jnp.int32) + jnp.int32(0x8000), 16)


def _routing_kernel(scores_ref, x_ref, pos0_ref, pos1_ref, pr0_ref, pr1_ref,
                    emap_ref, xpk_ref):
    xv = x_ref[...]
    lo = _bf16_bits(xv[:, :D2])
    hi = _bf16_bits(xv[:, D2:])
    xpk_ref[...] = jnp.bitwise_or(lo, lax.shift_left(hi, 16))
    scores = scores_ref[...]
    iota_e = lax.broadcasted_iota(jnp.int32, (S, E), 1)
    m1 = jnp.max(scores, axis=1, keepdims=True)
    i1 = jnp.min(jnp.where(scores == m1, iota_e, E), axis=1, keepdims=True)
    masked = jnp.where(iota_e == i1, -jnp.inf, scores)
    m2 = jnp.max(masked, axis=1, keepdims=True)
    i2 = jnp.min(jnp.where(masked == m2, iota_e, E), axis=1, keepdims=True)
    e2 = jnp.exp(m2 - m1)
    denom = 1.0 + e2
    pr0_ref[...] = 1.0 / denom
    pr1_ref[...] = e2 / denom

    one1 = (iota_e == i1).astype(jnp.float32)
    one2 = (iota_e == i2).astype(jnp.float32)
    osum = one1 + one2

    r = lax.broadcasted_iota(jnp.int32, (CHUNK, CHUNK), 0)
    c = lax.broadcasted_iota(jnp.int32, (CHUNK, CHUNK), 1)
    tril = (r > c).astype(jnp.float32)

    run = jnp.zeros((1, E), jnp.float32)
    rank1_chunks, rank2_chunks = [], []
    for ci in range(S // CHUNK):
        blk = osum[ci * CHUNK:(ci + 1) * CHUNK]
        cum = lax.dot_general(tril, blk, (((1,), (0,)), ((), ())),
                              preferred_element_type=jnp.float32) + run
        o1c = one1[ci * CHUNK:(ci + 1) * CHUNK]
        o2c = one2[ci * CHUNK:(ci + 1) * CHUNK]
        rank1_chunks.append(jnp.sum(cum * o1c, axis=1, keepdims=True))
        rank2_chunks.append(jnp.sum(cum * o2c, axis=1, keepdims=True))
        run = run + jnp.sum(blk, axis=0, keepdims=True)
    rank1 = jnp.concatenate(rank1_chunks, axis=0)
    rank2 = jnp.concatenate(rank2_chunks, axis=0)

    counts = run
    asz = jnp.floor((counts + (TM - 1)) * (1.0 / TM)) * TM
    ru = lax.broadcasted_iota(jnp.int32, (E, E), 0)
    cu = lax.broadcasted_iota(jnp.int32, (E, E), 1)
    upper = (ru < cu).astype(jnp.float32)
    starts = lax.dot_general(asz, upper, (((1,), (0,)), ((), ())),
                             preferred_element_type=jnp.float32)

    pos0 = jnp.sum(starts * one1, axis=1, keepdims=True) + rank1
    pos1 = jnp.sum(starts * one2, axis=1, keepdims=True) + rank2
    pos0_ref[...] = pos0.astype(jnp.int32)
    pos1_ref[...] = pos1.astype(jnp.int32)

    t_start = lax.broadcasted_iota(jnp.int32, (1, NTILES), 1).astype(
        jnp.float32) * TM
    lane8 = lax.broadcasted_iota(jnp.int32, (1, E), 1)
    acc = jnp.zeros((1, NTILES), jnp.float32)
    for e in range(E):
        s_e = jnp.sum(jnp.where(lane8 == e, starts, 0.0))
        acc = acc + (t_start >= s_e).astype(jnp.float32)
    emap_ref[...] = acc.astype(jnp.int32) - 1


def _dispatch_body(x_hbm, pos0_hbm, pos1_hbm, xs_hbm, rows_v, idx0_v, idx1_v,
                   sem0, sem1):
    wid = lax.axis_index("s") * 2 + lax.axis_index("c")
    base = wid * TOK_W
    pltpu.sync_copy(x_hbm.at[pl.ds(base, TOK_W)], rows_v)
    pltpu.sync_copy(pos0_hbm.at[pl.ds(base, TOK_W)], idx0_v)
    pltpu.sync_copy(pos1_hbm.at[pl.ds(base, TOK_W)], idx1_v)
    c0 = pltpu.async_copy(rows_v, xs_hbm.at[idx0_v], sem0)
    c1 = pltpu.async_copy(rows_v, xs_hbm.at[idx1_v], sem1)
    c0.wait()
    c1.wait()


def _ffn_kernel(emap_ref, xs_ref, w1_ref, w2_ref, w3_ref, out_ref,
                w1b_ref, w2b_ref, w3b_ref):
    i = pl.program_id(0)
    changed = jnp.logical_or(
        i == 0, emap_ref[i] != emap_ref[jnp.maximum(i - 1, 0)])

    @pl.when(changed)
    def _():
        w1b_ref[...] = w1_ref[0].astype(jnp.bfloat16)
        w2b_ref[...] = w2_ref[0].astype(jnp.bfloat16)
        w3b_ref[...] = w3_ref[0].astype(jnp.bfloat16)

    wv = xs_ref[...]
    xlo = lax.bitcast_convert_type(lax.shift_left(wv, 16), jnp.float32)
    xhi = lax.bitcast_convert_type(
        jnp.bitwise_and(wv, jnp.int32(_HI_MASK)), jnp.float32)
    xb = jnp.concatenate([xlo, xhi], axis=1).astype(jnp.bfloat16)
    a = lax.dot_general(xb, w1b_ref[...], (((1,), (1,)), ((), ())),
                        preferred_element_type=jnp.float32)
    b = lax.dot_general(xb, w2b_ref[...], (((1,), (1,)), ((), ())),
                        preferred_element_type=jnp.float32)
    h = (jax.nn.silu(a) * b).astype(jnp.bfloat16)
    o = lax.dot_general(h, w3b_ref[...], (((1,), (1,)), ((), ())),
                        preferred_element_type=jnp.float32)
    out_ref[...] = jnp.bitwise_or(
        _bf16_bits(o[:, :D2]), lax.shift_left(_bf16_bits(o[:, D2:]), 16))


_ffn_grid_spec = pltpu.PrefetchScalarGridSpec(
    num_scalar_prefetch=1,
    grid=(NTILES,),
    in_specs=[
        pl.BlockSpec((TM, D2), lambda i, em: (i, 0)),
        pl.BlockSpec((1, DFF, D), lambda i, em: (em[i], 0, 0)),
        pl.BlockSpec((1, DFF, D), lambda i, em: (em[i], 0, 0)),
        pl.BlockSpec((1, D, DFF), lambda i, em: (em[i], 0, 0)),
    ],
    out_specs=pl.BlockSpec((TM, D2), lambda i, em: (i, 0)),
    scratch_shapes=[
        pltpu.VMEM((DFF, D), jnp.bfloat16),
        pltpu.VMEM((DFF, D), jnp.bfloat16),
        pltpu.VMEM((D, DFF), jnp.bfloat16),
    ],
)


def _bf16_pair_fma(r0, r1, p0, p1):
    hi_mask = jnp.int32(_HI_MASK)
    _f = lambda v: lax.bitcast_convert_type(v, jnp.float32)
    lo0 = _f(lax.shift_left(r0, 16))
    lo1 = _f(lax.shift_left(r1, 16))
    hi0 = _f(jnp.bitwise_and(r0, hi_mask))
    hi1 = _f(jnp.bitwise_and(r1, hi_mask))
    return lo0 * p0 + lo1 * p1, hi0 * p0 + hi1 * p1


def _combine_body(outp_hbm, pos0_hbm, pos1_hbm, pr0_hbm, pr1_hbm, y_hbm,
                  rows0_a, rows1_a, rows0_b, rows1_b, y_v,
                  idx0_v, idx1_v, pr0_v, pr1_v, sem0, sem1, sem_y):
    wid = lax.axis_index("s") * 2 + lax.axis_index("c")
    base = wid * TOK_W
    pltpu.sync_copy(pos0_hbm.at[pl.ds(base, TOK_W)], idx0_v)
    pltpu.sync_copy(pos1_hbm.at[pl.ds(base, TOK_W)], idx1_v)
    pltpu.sync_copy(pr0_hbm.at[pl.ds(base, TOK_W)], pr0_v)
    pltpu.sync_copy(pr1_hbm.at[pl.ds(base, TOK_W)], pr1_v)

    rows0 = [rows0_a, rows0_b]
    rows1 = [rows1_a, rows1_b]

    def issue(c):
        i0 = idx0_v.at[pl.ds(c * TOK_C, TOK_C)]
        i1 = idx1_v.at[pl.ds(c * TOK_C, TOK_C)]
        h0 = pltpu.async_copy(outp_hbm.at[i0], rows0[c], sem0)
        h1 = pltpu.async_copy(outp_hbm.at[i1], rows1[c], sem1)
        return h0, h1

    nch = TOK_W // TOK_C
    handles = {0: issue(0)}
    ywrite = None
    for c in range(nch):
        handles[c][0].wait()
        handles[c][1].wait()
        if c + 1 < nch:
            handles[c + 1] = issue(c + 1)
        if ywrite is not None:
            ywrite.wait()
        for g in range(TOK_C // 16):
            prv0 = pr0_v[pl.ds(c * TOK_C + g * 16, 16)]
            prv1 = pr1_v[pl.ds(c * TOK_C + g * 16, 16)]
            for tl in range(16):
                t = g * 16 + tl
                lane = jnp.full((16,), tl, jnp.int32)
                p0 = prv0[lane]
                p1 = prv1[lane]

                def body(ci, carry, t=t, p0=p0, p1=p1, c=c):
                    sl = pl.ds(ci * 16, 16)
                    ylo, yhi = _bf16_pair_fma(
                        rows0[c][t, sl], rows1[c][t, sl], p0, p1)
                    y_v[t, pl.ds(ci * 16, 16)] = ylo
                    y_v[t, pl.ds(D2 + ci * 16, 16)] = yhi
                    return carry

                lax.fori_loop(0, D2 // 16, body, 0)
        ywrite = pltpu.async_copy(
            y_v, y_hbm.at[pl.ds(base + c * TOK_C, TOK_C)], sem_y)
    ywrite.wait()


@functools.cache
def _sc_kernels():
    mesh = plsc.VectorSubcoreMesh(core_axis_name="c", subcore_axis_name="s")
    dispatch = pl.kernel(
        _dispatch_body,
        out_type=jax.ShapeDtypeStruct((NP, D2), jnp.int32),
        mesh=mesh,
        scratch_types=[
            pltpu.VMEM((TOK_W, D2), jnp.int32),
            pltpu.VMEM((TOK_W,), jnp.int32),
            pltpu.VMEM((TOK_W,), jnp.int32),
            pltpu.SemaphoreType.DMA,
            pltpu.SemaphoreType.DMA,
        ],
    )
    combine = pl.kernel(
        _combine_body,
        out_type=jax.ShapeDtypeStruct((S, D), jnp.float32),
        mesh=mesh,
        scratch_types=[
            pltpu.VMEM((TOK_C, D2), jnp.int32),
            pltpu.VMEM((TOK_C, D2), jnp.int32),
            pltpu.VMEM((TOK_C, D2), jnp.int32),
            pltpu.VMEM((TOK_C, D2), jnp.int32),
            pltpu.VMEM((TOK_C, D), jnp.float32),
            pltpu.VMEM((TOK_W,), jnp.int32),
            pltpu.VMEM((TOK_W,), jnp.int32),
            pltpu.VMEM((TOK_W,), jnp.float32),
            pltpu.VMEM((TOK_W,), jnp.float32),
            pltpu.SemaphoreType.DMA,
            pltpu.SemaphoreType.DMA,
            pltpu.SemaphoreType.DMA,
        ],
    )
    return dispatch, combine


@jax.jit
def kernel(x, Wg, W1, W2, W3):
    x2d = x.reshape(S, D)
    scores = jnp.einsum('bsd,ed->bse', x, Wg).reshape(S, E)

    pos0, pos1, pr0, pr1, emap, xpk = pl.pallas_call(
        _routing_kernel,
        out_shape=[
            jax.ShapeDtypeStruct((S, 1), jnp.int32),
            jax.ShapeDtypeStruct((S, 1), jnp.int32),
            jax.ShapeDtypeStruct((S, 1), jnp.float32),
            jax.ShapeDtypeStruct((S, 1), jnp.float32),
            jax.ShapeDtypeStruct((1, NTILES), jnp.int32),
            jax.ShapeDtypeStruct((S, D2), jnp.int32),
        ],
    )(scores, x2d)
    pos0f = pos0.reshape(S)
    pos1f = pos1.reshape(S)

    dispatch, combine = _sc_kernels()
    xs_pk = dispatch(xpk, pos0f, pos1f)
    outp_pk = pl.pallas_call(
        _ffn_kernel,
        grid_spec=_ffn_grid_spec,
        out_shape=jax.ShapeDtypeStruct((NP, D2), jnp.int32),
    )(emap.reshape(NTILES), xs_pk, W1, W2, W3)
    y = combine(outp_pk, pos0f, pos1f, pr0.reshape(S), pr1.reshape(S))
    return y.reshape(B, S, D)
